# baseline (speedup 1.0000x reference)
import jax
import jax.numpy as jnp
from jax.experimental import pallas as pl
from jax.experimental.pallas import tpu as pltpu
from jax.experimental.pallas import tpu_sc as plsc

_BLKR = 128


def kernel(x, pos_table):
    B, S, D = x.shape
    xt = jnp.transpose(x, (1, 0, 2)).reshape(S * B, D)
    nblk = (S * B) // _BLKR
    per_s = B // _BLKR

    @pl.kernel(
        out_type=jax.ShapeDtypeStruct((S * B, D), x.dtype),
        mesh=plsc.VectorSubcoreMesh(core_axis_name="c", subcore_axis_name="s"),
    )
    def sc_kernel(x_hbm, pos_hbm, o_hbm):
        def body(x_vmem, pos_vmem, o_vmem):
            @pl.loop(0, _BLKR)
            def _(r):
                for d in range(0, D, 16):
                    slc = (pl.ds(r, 1), pl.ds(d, 16))
                    o_vmem.at[*slc][...] = (
                        x_vmem.at[*slc][...]
                        + pos_vmem.at[pl.ds(0, 1), pl.ds(d, 16)][...]
                    )

        pltpu.emit_pipeline(
            body,
            grid=(nblk,),
            in_specs=[
                pl.BlockSpec((_BLKR, D), lambda i: (i, 0)),
                pl.BlockSpec((1, D), lambda i: (i // per_s, 0)),
            ],
            out_specs=[pl.BlockSpec((_BLKR, D), lambda i: (i, 0))],
            core_axis_name=("c", "s"),
            dimension_semantics=(pltpu.PARALLEL,),
        )(x_hbm, pos_hbm, o_hbm)

    out2 = sc_kernel(xt, pos_table)
    return jnp.transpose(out2.reshape(S, B, D), (1, 0, 2))


# speedup vs baseline: 4.3505x; 4.3505x over previous
import jax
import jax.numpy as jnp
from jax.experimental import pallas as pl

_SB = 24


def _posadd_kernel(x_ref, pos_ref, o_ref):
    i = pl.program_id(0)
    pos = pos_ref[pl.ds(i * _SB, _SB), :]
    o_ref[...] = x_ref[...] + pos[:, None, :]


def kernel(x, pos_table):
    B, S, D = x.shape
    xt = jnp.transpose(x, (1, 0, 2))
    out_t = pl.pallas_call(
        _posadd_kernel,
        grid=(pl.cdiv(S, _SB),),
        in_specs=[
            pl.BlockSpec((_SB, B, D), lambda i: (i, 0, 0)),
            pl.BlockSpec((512, D), lambda i: (0, 0)),
        ],
        out_specs=pl.BlockSpec((_SB, B, D), lambda i: (i, 0, 0)),
        out_shape=jax.ShapeDtypeStruct((S, B, D), x.dtype),
    )(xt, pos_table)
    return jnp.transpose(out_t, (1, 0, 2))
